# initial kernel scaffold (unmeasured)
import jax
import jax.numpy as jnp
from jax import lax
from jax.experimental import pallas as pl
from jax.experimental.pallas import tpu as pltpu


def kernel(
    x,
):
    def body(*refs):
        pass

    out_shape = jax.ShapeDtypeStruct(..., jnp.float32)
    return pl.pallas_call(body, out_shape=out_shape)(...)


# baseline (device time: 156840 ns/iter reference)
import jax
import jax.numpy as jnp
from jax import lax
from jax.experimental import pallas as pl
from jax.experimental.pallas import tpu as pltpu

M, N = 2048, 1024
SY, SZ = 4, 4
N_PHASES = 6


def kernel(x):
    x2 = x.reshape(M, N)

    def body(x_ref, out_ref, send_buf, recv_buf, send_sem, recv_sem, credit_sems):
        mx = lax.axis_index("x")
        my = lax.axis_index("y")
        mz = lax.axis_index("z")

        xo = 1 - mx
        yl, yr = (my - 1) % SY, (my + 1) % SY
        zl, zr = (mz - 1) % SZ, (mz + 1) % SZ

        x_dev = (xo, my, mz)
        yl_dev, yr_dev = (mx, yl, mz), (mx, yr, mz)
        zl_dev, zr_dev = (mx, my, zl), (mx, my, zr)

        barrier = pltpu.get_barrier_semaphore()
        for dev in (x_dev, yl_dev, yr_dev, zl_dev, zr_dev):
            pl.semaphore_signal(
                barrier, inc=1, device_id=dev,
                device_id_type=pl.DeviceIdType.MESH,
            )
        pl.semaphore_wait(barrier, 5)

        out_ref[...] = x_ref[...]

        first = [True]

        def step(P, rows, send_off, recv_off, right_dev, accumulate, credits):
            send_buf[pl.ds(0, rows), :] = out_ref[
                pl.ds(send_off, rows), :
            ].astype(jnp.bfloat16)
            if first[0]:
                first[0] = False
            else:
                pl.semaphore_wait(credit_sems.at[P], 1)
            rdma = pltpu.make_async_remote_copy(
                src_ref=send_buf.at[pl.ds(0, rows), :],
                dst_ref=recv_buf.at[pl.ds(0, rows), :],
                send_sem=send_sem,
                recv_sem=recv_sem,
                device_id=right_dev,
                device_id_type=pl.DeviceIdType.MESH,
            )
            rdma.start()
            rdma.wait()
            inc = recv_buf[pl.ds(0, rows), :].astype(jnp.float32)
            if accumulate:
                out_ref[pl.ds(recv_off, rows), :] = (
                    out_ref[pl.ds(recv_off, rows), :] + inc
                )
            else:
                out_ref[pl.ds(recv_off, rows), :] = inc
            for ci, dev in credits:
                pl.semaphore_signal(
                    credit_sems.at[ci], inc=1, device_id=dev,
                    device_id_type=pl.DeviceIdType.MESH,
                )

        step(0, 1024, mx * 1024, xo * 1024, x_dev, True, [(1, yl_dev)])
        off_x = xo * 1024

        for h in range(SY - 1):
            sc = (my - h) % SY
            rc = (my - h - 1) % SY
            credits = [(1, yl_dev)] if h < SY - 2 else [(2, zl_dev)]
            step(1, 256, off_x + sc * 256, off_x + rc * 256, yr_dev, True, credits)
        off_xy = off_x + ((my + 1) % SY) * 256

        for h in range(SZ - 1):
            sc = (mz - h) % SZ
            rc = (mz - h - 1) % SZ
            credits = [(2, zl_dev)] if h < SZ - 2 else [(3, zl_dev)]
            step(2, 64, off_xy + sc * 64, off_xy + rc * 64, zr_dev, True, credits)

        for h in range(SZ - 1):
            sc = (mz + 1 - h) % SZ
            rc = (mz - h) % SZ
            credits = [(3, zl_dev)] if h < SZ - 2 else [(4, yl_dev)]
            step(3, 64, off_xy + sc * 64, off_xy + rc * 64, zr_dev, False, credits)

        for h in range(SY - 1):
            sc = (my + 1 - h) % SY
            rc = (my - h) % SY
            credits = [(4, yl_dev)] if h < SY - 2 else [(5, x_dev)]
            step(4, 256, off_x + sc * 256, off_x + rc * 256, yr_dev, False, credits)

        step(5, 1024, xo * 1024, mx * 1024, x_dev, False, [])

    return pl.pallas_call(
        body,
        out_shape=jax.ShapeDtypeStruct((M, N), jnp.float32),
        in_specs=[pl.BlockSpec(memory_space=pltpu.VMEM)],
        out_specs=pl.BlockSpec(memory_space=pltpu.VMEM),
        scratch_shapes=[
            pltpu.VMEM((1024, N), jnp.bfloat16),
            pltpu.VMEM((1024, N), jnp.bfloat16),
            pltpu.SemaphoreType.DMA,
            pltpu.SemaphoreType.DMA,
            pltpu.SemaphoreType.REGULAR((N_PHASES,)),
        ],
        compiler_params=pltpu.CompilerParams(collective_id=0),
    )(x2)


# device time: 114353 ns/iter; 1.3715x vs baseline; 1.3715x over previous
import jax
import jax.numpy as jnp
from jax import lax
from jax.experimental import pallas as pl
from jax.experimental.pallas import tpu as pltpu

M, N = 2048, 1024
HALF = M // 2
SY, SZ = 4, 4
N_PHASES = 6


def kernel(x):
    x2 = x.reshape(M, N)

    def body(x_ref, out_ref,
             send_a, recv_a, send_b, recv_b,
             ssem_a, rsem_a, ssem_b, rsem_b,
             cred_a, cred_b):
        mx = lax.axis_index("x")
        my = lax.axis_index("y")
        mz = lax.axis_index("z")

        xo = 1 - mx
        yl, yr = (my - 1) % SY, (my + 1) % SY
        zl, zr = (mz - 1) % SZ, (mz + 1) % SZ

        x_dev = (xo, my, mz)
        yl_dev, yr_dev = (mx, yl, mz), (mx, yr, mz)
        zl_dev, zr_dev = (mx, my, zl), (mx, my, zr)

        barrier = pltpu.get_barrier_semaphore()
        for dev in (x_dev, yl_dev, yr_dev, zl_dev, zr_dev):
            pl.semaphore_signal(
                barrier, inc=1, device_id=dev,
                device_id_type=pl.DeviceIdType.MESH,
            )
        pl.semaphore_wait(barrier, 5)

        y_ring = (my, SY, yr_dev, yl_dev)
        z_ring = (mz, SZ, zr_dev, zl_dev)
        x_ring = (mx, 2, x_dev, x_dev)

        def build_half(base, ring1, ring2):
            p1, S1, r1_dev, l1_dev = ring1
            p2, S2, r2_dev, l2_dev = ring2
            c1 = HALF // S1
            c2 = c1 // S2
            c3 = c2 // 2
            steps = []
            for h in range(S1 - 1):
                sc = (p1 - h) % S1
                rc = (p1 - h - 1) % S1
                steps.append((0, c1, base + sc * c1, base + rc * c1,
                              r1_dev, 'add_x' if h == 0 else 'add_x_fwd'))
            off1 = base + ((p1 + 1) % S1) * c1
            for h in range(S2 - 1):
                sc = (p2 - h) % S2
                rc = (p2 - h - 1) % S2
                steps.append((1, c2, off1 + sc * c2, off1 + rc * c2,
                              r2_dev, 'add'))
            off2 = off1 + ((p2 + 1) % S2) * c2
            steps.append((2, c3, off2 + mx * c3, off2 + xo * c3,
                          x_dev, 'add'))
            steps.append((3, c3, off2 + xo * c3, off2 + mx * c3,
                          x_dev, 'store'))
            for h in range(S2 - 1):
                sc = (p2 + 1 - h) % S2
                rc = (p2 - h) % S2
                steps.append((4, c2, off1 + sc * c2, off1 + rc * c2,
                              r2_dev, 'store'))
            for h in range(S1 - 1):
                sc = (p1 + 1 - h) % S1
                rc = (p1 - h) % S1
                steps.append((5, c1, base + sc * c1, base + rc * c1,
                              r1_dev, 'store'))
            lefts = {0: l1_dev, 1: l2_dev, 2: x_dev, 3: x_dev,
                     4: l2_dev, 5: l1_dev}
            plan = []
            for k, st in enumerate(steps):
                wait_credit = k > 0
                if k + 1 < len(steps):
                    nxt = steps[k + 1][0]
                    credit_sig = (nxt, lefts[nxt])
                else:
                    credit_sig = None
                plan.append(st + (wait_credit, credit_sig))
            return plan

        plan_a = build_half(0, y_ring, z_ring)
        plan_b = build_half(HALF, z_ring, y_ring)
        assert len(plan_a) == len(plan_b)

        def stage(st, send_buf):
            phase, rows, soff, _roff, _dev, mode, _w, _c = st
            src = x_ref if mode == 'add_x' else out_ref
            send_buf[pl.ds(0, rows), :] = src[pl.ds(soff, rows), :].astype(
                jnp.bfloat16)

        def launch(st, send_buf, recv_buf, ssem, rsem, cred):
            phase, rows, _soff, _roff, dev, _m, wait_credit, _c = st
            if wait_credit:
                pl.semaphore_wait(cred.at[phase], 1)
            rdma = pltpu.make_async_remote_copy(
                src_ref=send_buf.at[pl.ds(0, rows), :],
                dst_ref=recv_buf.at[pl.ds(0, rows), :],
                send_sem=ssem,
                recv_sem=rsem,
                device_id=dev,
                device_id_type=pl.DeviceIdType.MESH,
            )
            rdma.start()
            return rdma

        def consume(st, recv_buf, cred):
            phase, rows, _soff, roff, _dev, mode, _w, credit_sig = st
            inc = recv_buf[pl.ds(0, rows), :].astype(jnp.float32)
            if mode in ('add_x', 'add_x_fwd'):
                out_ref[pl.ds(roff, rows), :] = (
                    x_ref[pl.ds(roff, rows), :] + inc)
            elif mode == 'add':
                out_ref[pl.ds(roff, rows), :] = (
                    out_ref[pl.ds(roff, rows), :] + inc)
            else:
                out_ref[pl.ds(roff, rows), :] = inc
            if credit_sig is not None:
                ci, dev = credit_sig
                pl.semaphore_signal(
                    cred.at[ci], inc=1, device_id=dev,
                    device_id_type=pl.DeviceIdType.MESH,
                )

        for sa, sb in zip(plan_a, plan_b):
            stage(sa, send_a)
            stage(sb, send_b)
            ra = launch(sa, send_a, recv_a, ssem_a, rsem_a, cred_a)
            rb = launch(sb, send_b, recv_b, ssem_b, rsem_b, cred_b)
            ra.wait()
            rb.wait()
            consume(sa, recv_a, cred_a)
            consume(sb, recv_b, cred_b)

    return pl.pallas_call(
        body,
        out_shape=jax.ShapeDtypeStruct((M, N), jnp.float32),
        in_specs=[pl.BlockSpec(memory_space=pltpu.VMEM)],
        out_specs=pl.BlockSpec(memory_space=pltpu.VMEM),
        scratch_shapes=[
            pltpu.VMEM((HALF // SY, N), jnp.bfloat16),
            pltpu.VMEM((HALF // SY, N), jnp.bfloat16),
            pltpu.VMEM((HALF // SZ, N), jnp.bfloat16),
            pltpu.VMEM((HALF // SZ, N), jnp.bfloat16),
            pltpu.SemaphoreType.DMA,
            pltpu.SemaphoreType.DMA,
            pltpu.SemaphoreType.DMA,
            pltpu.SemaphoreType.DMA,
            pltpu.SemaphoreType.REGULAR((N_PHASES,)),
            pltpu.SemaphoreType.REGULAR((N_PHASES,)),
        ],
        compiler_params=pltpu.CompilerParams(collective_id=0),
    )(x2)


# device time: 86930 ns/iter; 1.8042x vs baseline; 1.3155x over previous
import jax
import jax.numpy as jnp
from jax import lax
from jax.experimental import pallas as pl
from jax.experimental.pallas import tpu as pltpu

M, N = 2048, 1024
HALF = M // 2
S4 = 4
C1 = HALF // S4
C2 = C1 // S4
C3 = C2 // 2
N_PHASES = 6


def kernel(x):
    x2 = x.reshape(M, N)

    def body(x_ref, out_ref,
             send_a, recv_a, send_b, recv_b,
             ssem_a, rsem_a, ssem_b, rsem_b,
             cred_a, cred_b):
        mx = lax.axis_index("x")
        my = lax.axis_index("y")
        mz = lax.axis_index("z")
        xo = 1 - mx

        x_dev = (xo, my, mz)

        def y_dev(j):
            return (mx, j, mz)

        def z_dev(j):
            return (mx, my, j)

        y_ring = (my, y_dev)
        z_ring = (mz, z_dev)

        barrier = pltpu.get_barrier_semaphore()
        for d in range(1, S4):
            pl.semaphore_signal(
                barrier, inc=1, device_id=y_dev((my + d) % S4),
                device_id_type=pl.DeviceIdType.MESH,
            )
            pl.semaphore_signal(
                barrier, inc=1, device_id=z_dev((mz + d) % S4),
                device_id_type=pl.DeviceIdType.MESH,
            )
        pl.semaphore_wait(barrier, 2 * (S4 - 1))

        f32 = jnp.float32
        bf16 = jnp.bfloat16

        def make_half(base, ring1, ring2, send_buf, recv_buf, ssems, rsems, cred):
            p1, dev1 = ring1
            p2, dev2 = ring2
            off1 = base + p1 * C1
            off2 = off1 + p2 * C2
            off3 = off2 + mx * C3

            def mates(p, dev):
                return [dev((p + d) % S4) for d in range(1, S4)]

            def sig(ci, devs):
                for dv in devs:
                    pl.semaphore_signal(
                        cred.at[ci], inc=1, device_id=dv,
                        device_id_type=pl.DeviceIdType.MESH,
                    )

            def starts(rows, targets):
                rdmas = []
                for slot, dv in targets:
                    r = pltpu.make_async_remote_copy(
                        src_ref=send_buf.at[slot, pl.ds(0, rows), :],
                        dst_ref=recv_buf.at[slot, pl.ds(0, rows), :],
                        send_sem=ssems.at[slot],
                        recv_sem=rsems.at[slot],
                        device_id=dv,
                        device_id_type=pl.DeviceIdType.MESH,
                    )
                    r.start()
                    rdmas.append(r)
                return rdmas

            def rs4(P, p, dev, boff, c, src_is_x, nxt_devs):
                src = x_ref if src_is_x else out_ref

                def stage():
                    for d in range(1, S4):
                        q = (p + d) % S4
                        send_buf[d - 1, pl.ds(0, c), :] = src[
                            pl.ds(boff + q * c, c), :].astype(bf16)

                def credit_wait():
                    if P > 0:
                        pl.semaphore_wait(cred.at[P], S4 - 1)

                def start():
                    return starts(c, [(d - 1, dev((p + d) % S4))
                                      for d in range(1, S4)])

                def consume():
                    own = boff + p * c
                    acc = src[pl.ds(own, c), :]
                    for d in range(1, S4):
                        acc = acc + recv_buf[d - 1, pl.ds(0, c), :].astype(f32)
                    out_ref[pl.ds(own, c), :] = acc

                def credit_sig():
                    sig(P + 1, nxt_devs)

                return stage, credit_wait, start, consume, credit_sig

            def ag4(P, p, dev, boff, c, nxt_devs):
                def stage():
                    send_buf[0, pl.ds(0, c), :] = out_ref[
                        pl.ds(boff + p * c, c), :].astype(bf16)

                def credit_wait():
                    pl.semaphore_wait(cred.at[P], S4 - 1)

                def start():
                    rdmas = []
                    for d in range(1, S4):
                        r = pltpu.make_async_remote_copy(
                            src_ref=send_buf.at[0, pl.ds(0, c), :],
                            dst_ref=recv_buf.at[d - 1, pl.ds(0, c), :],
                            send_sem=ssems.at[d - 1],
                            recv_sem=rsems.at[d - 1],
                            device_id=dev((p + d) % S4),
                            device_id_type=pl.DeviceIdType.MESH,
                        )
                        r.start()
                        rdmas.append(r)
                    return rdmas

                def consume():
                    for d in range(1, S4):
                        s = (p - d) % S4
                        out_ref[pl.ds(boff + s * c, c), :] = recv_buf[
                            d - 1, pl.ds(0, c), :].astype(f32)

                def credit_sig():
                    if nxt_devs is not None:
                        sig(P + 1, nxt_devs)

                return stage, credit_wait, start, consume, credit_sig

            def x_rs(P, nxt_devs):
                def stage():
                    send_buf[0, pl.ds(0, C3), :] = out_ref[
                        pl.ds(off2 + xo * C3, C3), :].astype(bf16)

                def credit_wait():
                    pl.semaphore_wait(cred.at[P], 1)

                def start():
                    return starts(C3, [(0, x_dev)])

                def consume():
                    own = off2 + mx * C3
                    out_ref[pl.ds(own, C3), :] = (
                        out_ref[pl.ds(own, C3), :]
                        + recv_buf[0, pl.ds(0, C3), :].astype(f32))

                def credit_sig():
                    sig(P + 1, [x_dev])

                return stage, credit_wait, start, consume, credit_sig

            def x_ag(P, nxt_devs):
                def stage():
                    send_buf[0, pl.ds(0, C3), :] = out_ref[
                        pl.ds(off3, C3), :].astype(bf16)

                def credit_wait():
                    pl.semaphore_wait(cred.at[P], 1)

                def start():
                    return starts(C3, [(0, x_dev)])

                def consume():
                    out_ref[pl.ds(off2 + xo * C3, C3), :] = recv_buf[
                        0, pl.ds(0, C3), :].astype(f32)

                def credit_sig():
                    sig(P + 1, nxt_devs)

                return stage, credit_wait, start, consume, credit_sig

            m1 = mates(p1, dev1)
            m2 = mates(p2, dev2)
            return [
                rs4(0, p1, dev1, base, C1, True, m2),
                rs4(1, p2, dev2, off1, C2, False, [x_dev]),
                x_rs(2, [x_dev]),
                x_ag(3, m2),
                ag4(4, p2, dev2, off1, C2, m1),
                ag4(5, p1, dev1, base, C1, None),
            ]

        half_a = make_half(0, y_ring, z_ring, send_a, recv_a,
                           ssem_a, rsem_a, cred_a)
        half_b = make_half(HALF, z_ring, y_ring, send_b, recv_b,
                           ssem_b, rsem_b, cred_b)

        for pa, pb in zip(half_a, half_b):
            st_a, cw_a, go_a, con_a, cs_a = pa
            st_b, cw_b, go_b, con_b, cs_b = pb
            st_a()
            st_b()
            cw_a()
            ras = go_a()
            cw_b()
            rbs = go_b()
            for r in ras:
                r.wait()
            for r in rbs:
                r.wait()
            con_a()
            con_b()
            cs_a()
            cs_b()

    return pl.pallas_call(
        body,
        out_shape=jax.ShapeDtypeStruct((M, N), jnp.float32),
        in_specs=[pl.BlockSpec(memory_space=pltpu.VMEM)],
        out_specs=pl.BlockSpec(memory_space=pltpu.VMEM),
        scratch_shapes=[
            pltpu.VMEM((3, C1, N), jnp.bfloat16),
            pltpu.VMEM((3, C1, N), jnp.bfloat16),
            pltpu.VMEM((3, C1, N), jnp.bfloat16),
            pltpu.VMEM((3, C1, N), jnp.bfloat16),
            pltpu.SemaphoreType.DMA((3,)),
            pltpu.SemaphoreType.DMA((3,)),
            pltpu.SemaphoreType.DMA((3,)),
            pltpu.SemaphoreType.DMA((3,)),
            pltpu.SemaphoreType.REGULAR((N_PHASES,)),
            pltpu.SemaphoreType.REGULAR((N_PHASES,)),
        ],
        compiler_params=pltpu.CompilerParams(collective_id=0),
    )(x2)
